# baseline (speedup 1.0000x reference)
import functools

import jax
import jax.numpy as jnp
from jax import lax
from jax.experimental import pallas as pl
from jax.experimental.pallas import tpu as pltpu
from jax.experimental.pallas import tpu_sc as plsc

N = 4096
K = 20
NC = 4
CLA = 10


_BR = 128
_CW = 512
_NCHK = N // _CW


def _knn_body(pos_ref, posT3_ref, batch_ref, batchT_ref, batchT3_ref,
              idx_ref):
    b = batch_ref[...]
    bt = batchT_ref[...]
    bmin = jnp.min(b)
    bmax = jnp.max(b)
    col1 = lax.broadcasted_iota(jnp.int32, (1, N), 1)
    colstart = jnp.min(jnp.where(bt >= bmin, col1, N))
    colend = jnp.max(jnp.where(bt <= bmax, col1, -1)) + 1

    inf = jnp.float32(jnp.inf)

    def fast(nc):
        cs = jnp.minimum(colstart // _CW, _NCHK - nc)
        base = cs * _CW
        parts = []
        for i in range(nc):
            pj = posT3_ref[cs + i]
            d = jnp.zeros((_BR, _CW), jnp.float32)
            for cc in range(3):
                diff = pos_ref[:, cc : cc + 1] - pj[cc : cc + 1, :]
                d = d + diff * diff
            parts.append(jnp.where(b != batchT3_ref[cs + i], inf, d))
        d = jnp.concatenate(parts, axis=1)
        w = nc * _CW
        col = lax.broadcasted_iota(jnp.int32, (_BR, w), 1)
        outs = []
        for _ in range(K):
            m = jnp.min(d, axis=1, keepdims=True)
            sel = jnp.where(d == m, col, w)
            j = jnp.min(sel, axis=1, keepdims=True)
            outs.append(j + base)
            d = jnp.where(col == j, inf, d)
        idx_ref[...] = jnp.concatenate(outs, axis=1)

    cs3 = jnp.minimum(colstart // _CW, _NCHK - 3)
    fit3 = colend <= cs3 * _CW + 3 * _CW
    cs5 = jnp.minimum(colstart // _CW, _NCHK - 5)
    fit5 = colend <= cs5 * _CW + 5 * _CW

    @pl.when(fit3)
    def _():
        fast(3)

    @pl.when(jnp.logical_and(jnp.logical_not(fit3), fit5))
    def _():
        fast(5)

    @pl.when(jnp.logical_not(fit5))
    def _():
        fast(_NCHK)


def _knn(pos, batch):
    posT3 = pos.T.reshape(3, _NCHK, _CW).transpose(1, 0, 2)
    batch2d = batch.reshape(N, 1)
    batchT = batch.reshape(1, N)
    batchT3 = batch.reshape(_NCHK, 1, _CW)
    return pl.pallas_call(
        _knn_body,
        grid=(N // _BR,),
        in_specs=[
            pl.BlockSpec((_BR, 3), lambda i: (i, 0)),
            pl.BlockSpec((_NCHK, 3, _CW), lambda i: (0, 0, 0)),
            pl.BlockSpec((_BR, 1), lambda i: (i, 0)),
            pl.BlockSpec((1, N), lambda i: (0, 0)),
            pl.BlockSpec((_NCHK, 1, _CW), lambda i: (0, 0, 0)),
        ],
        out_specs=pl.BlockSpec((_BR, K), lambda i: (i, 0)),
        out_shape=jax.ShapeDtypeStruct((N, K), jnp.int32),
    )(pos, posT3, batch2d, batchT, batchT3)


_NW = 32
_B = K * N
_BPW = _B // _NW
_CH = 128
_NCH = _BPW // _CH


_D = 128


@jax.jit
def _gather_rows(table, idx3d):
    mesh = plsc.VectorSubcoreMesh(core_axis_name="c", subcore_axis_name="s")

    @functools.partial(
        pl.kernel,
        mesh=mesh,
        out_type=jax.ShapeDtypeStruct((_B, _D), jnp.float32),
        scratch_types=[
            pltpu.VMEM((_NCH, _CH), jnp.int32),
            pltpu.VMEM((2, _CH, _D), jnp.float32),
            pltpu.SemaphoreType.DMA,
            pltpu.SemaphoreType.DMA,
        ],
    )
    def gk(table_hbm, idx_hbm, out_hbm, idx_v, rows_v, sem0, sem1):
        wid = lax.axis_index("s") * 2 + lax.axis_index("c")
        base = wid * _BPW
        pltpu.sync_copy(idx_hbm.at[wid], idx_v)
        sems = (sem0, sem1)
        copies = [None, None]
        copies[0] = pltpu.async_copy(
            table_hbm.at[idx_v.at[0]], rows_v.at[0], sems[0])
        for j in range(_NCH):
            if j + 1 < _NCH:
                copies[(j + 1) % 2] = pltpu.async_copy(
                    table_hbm.at[idx_v.at[j + 1]], rows_v.at[(j + 1) % 2],
                    sems[(j + 1) % 2])
            copies[j % 2].wait()
            pltpu.sync_copy(rows_v.at[j % 2],
                            out_hbm.at[pl.ds(base + j * _CH, _CH)])

    return gk(table, idx3d)


_BT = 2048


def _edge_body(*refs, nlayer, d_real, pad_out):
    if nlayer == 3:
        (f_ref, g_ref, w1_ref, b1_ref, w2_ref, b2_ref, w3_ref,
         b3_ref, out_ref) = refs
    else:
        (f_ref, g_ref, w1_ref, b1_ref, w2_ref, b2_ref,
         out_ref) = refs
    f = f_ref[:, :d_real]
    w1 = w1_ref[...]
    acc = None
    for k in range(K):
        xj = g_ref[k][:, :d_real]
        e = jnp.concatenate([f, xj - f], axis=1)
        h = jnp.dot(e, w1, preferred_element_type=jnp.float32) + b1_ref[...]
        h = jnp.maximum(h, 0.0)
        h = jnp.dot(h, w2_ref[...], preferred_element_type=jnp.float32) + b2_ref[...]
        h = jnp.maximum(h, 0.0)
        if nlayer == 3:
            h = jnp.dot(h, w3_ref[...], preferred_element_type=jnp.float32) + b3_ref[...]
            h = jnp.maximum(h, 0.0)
        acc = h if acc is None else jnp.maximum(acc, h)
    if pad_out:
        acc = jnp.concatenate(
            [acc, jnp.zeros((acc.shape[0], _D - acc.shape[1]), jnp.float32)],
            axis=1)
    out_ref[...] = acc


def _edge_conv(f, g, w1, layers, d_real, pad_out):
    b1 = layers[0][1].reshape(1, -1)
    w2, b2 = layers[1]
    b2 = b2.reshape(1, -1)
    extra = []
    if len(layers) == 3:
        w3, b3 = layers[2]
        b3 = b3.reshape(1, -1)
        extra = [w3, b3]
    h_out = _D if pad_out else layers[-1][0].shape[1]
    full = lambda t: (0, 0)
    in_specs = [
        pl.BlockSpec((_BT, _D), lambda t: (t, 0)),
        pl.BlockSpec((K, _BT, _D), lambda t: (0, t, 0)),
        pl.BlockSpec(w1.shape, full),
        pl.BlockSpec(b1.shape, full),
        pl.BlockSpec(w2.shape, full),
        pl.BlockSpec(b2.shape, full),
    ]
    if extra:
        in_specs += [pl.BlockSpec(extra[0].shape, full),
                     pl.BlockSpec(extra[1].shape, full)]
    return pl.pallas_call(
        functools.partial(_edge_body, nlayer=len(layers), d_real=d_real,
                          pad_out=pad_out),
        grid=(N // _BT,),
        in_specs=in_specs,
        out_specs=pl.BlockSpec((_BT, h_out), lambda t: (t, 0)),
        out_shape=jax.ShapeDtypeStruct((N, h_out), jnp.float32),
    )(f, g, w1, b1, w2, b2, *extra)


def _final_body(comb_ref, batchT_ref, wl1_ref, bl1_ref, wl2_ref, bl2_ref,
                wm_refs, out_ref):
    h = jnp.dot(comb_ref[...], wl1_ref[...], preferred_element_type=jnp.float32)
    h = jnp.maximum(h + bl1_ref[...], 0.0)
    h = jnp.dot(h, wl2_ref[...], preferred_element_type=jnp.float32) + bl2_ref[...]
    cls = lax.broadcasted_iota(jnp.int32, (NC, N), 0)
    oneh = (batchT_ref[...] == cls).astype(jnp.float32)
    pool = jnp.dot(oneh, h, preferred_element_type=jnp.float32,
                   precision=lax.Precision.HIGHEST)
    cnt = jnp.sum(oneh, axis=1, keepdims=True)
    m = pool / jnp.maximum(cnt, 1.0)
    nm = len(wm_refs) // 2
    for i in range(nm):
        m = jnp.dot(m, wm_refs[2 * i][...], preferred_element_type=jnp.float32)
        m = m + wm_refs[2 * i + 1][...]
        if i < nm - 1:
            m = jnp.maximum(m, 0.0)
    out_ref[...] = m


def _final(comb, batch, lin1, mlp):
    batchT = batch.reshape(1, N)
    wl1, bl1 = lin1[0]
    wl2, bl2 = lin1[1]
    args = [comb, batchT, wl1, bl1.reshape(1, -1), wl2, bl2.reshape(1, -1)]
    for w, b in mlp:
        args += [w, b.reshape(1, -1)]

    def body(*refs):
        _final_body(refs[0], refs[1], refs[2], refs[3], refs[4], refs[5],
                    refs[6:-1], refs[-1])

    return pl.pallas_call(
        body,
        out_shape=jax.ShapeDtypeStruct((NC, CLA), jnp.float32),
    )(*args)


def kernel(x, pos, tq, params, batch):
    del tq
    idx = _knn(pos, batch)
    idx3d = idx.T.reshape(_NW, _NCH, _CH)

    xx = jnp.concatenate(
        [x, pos, jnp.zeros((N, _D - 11), jnp.float32)], axis=1)
    g1 = _gather_rows(xx, idx3d).reshape(K, N, _D)
    x1 = _edge_conv(xx, g1, params['conv1'][0][0], params['conv1'], 11, True)

    g2 = _gather_rows(x1, idx3d).reshape(K, N, _D)
    x2 = _edge_conv(x1, g2, params['conv2'][0][0], params['conv2'], 64, True)

    g3 = _gather_rows(x2, idx3d).reshape(K, N, _D)
    x3 = _edge_conv(x2, g3, params['conv2'][0][0], params['conv2'], 64, True)

    g4 = _gather_rows(x3, idx3d).reshape(K, N, _D)
    x4 = _edge_conv(x3, g4, params['conv3'][0][0], params['conv3'], 64, False)

    comb = jnp.concatenate(
        [x1[:, :64], x2[:, :64], x3[:, :64], x4], axis=1)
    return _final(comb, batch, params['lin1'], params['mlp'])


# speedup vs baseline: 1.0348x; 1.0348x over previous
import functools

import jax
import jax.numpy as jnp
from jax import lax
from jax.experimental import pallas as pl
from jax.experimental.pallas import tpu as pltpu
from jax.experimental.pallas import tpu_sc as plsc

N = 4096
K = 20
NC = 4
CLA = 10


_BR = 128
_CW = 512
_NCHK = N // _CW


def _knn_body(pos_ref, posT3_ref, batch_ref, batchT_ref, batchT3_ref,
              idx_ref):
    b = batch_ref[...]
    bt = batchT_ref[...]
    bmin = jnp.min(b)
    bmax = jnp.max(b)
    col1 = lax.broadcasted_iota(jnp.int32, (1, N), 1)
    colstart = jnp.min(jnp.where(bt >= bmin, col1, N))
    colend = jnp.max(jnp.where(bt <= bmax, col1, -1)) + 1

    inf = jnp.float32(jnp.inf)

    def fast(nc):
        cs = jnp.minimum(colstart // _CW, _NCHK - nc)
        base = cs * _CW
        parts = []
        for i in range(nc):
            pj = posT3_ref[cs + i]
            d = jnp.zeros((_BR, _CW), jnp.float32)
            for cc in range(3):
                diff = pos_ref[:, cc : cc + 1] - pj[cc : cc + 1, :]
                d = d + diff * diff
            parts.append(jnp.where(b != batchT3_ref[cs + i], inf, d))
        d = jnp.concatenate(parts, axis=1)
        w = nc * _CW
        col = lax.broadcasted_iota(jnp.int32, (_BR, w), 1)
        outs = []
        for _ in range(K):
            m = jnp.min(d, axis=1, keepdims=True)
            sel = jnp.where(d == m, col, w)
            j = jnp.min(sel, axis=1, keepdims=True)
            outs.append(j + base)
            d = jnp.where(col == j, inf, d)
        idx_ref[...] = jnp.concatenate(outs, axis=1)

    cs3 = jnp.minimum(colstart // _CW, _NCHK - 3)
    fit3 = colend <= cs3 * _CW + 3 * _CW
    cs5 = jnp.minimum(colstart // _CW, _NCHK - 5)
    fit5 = colend <= cs5 * _CW + 5 * _CW

    @pl.when(fit3)
    def _():
        fast(3)

    @pl.when(jnp.logical_and(jnp.logical_not(fit3), fit5))
    def _():
        fast(5)

    @pl.when(jnp.logical_not(fit5))
    def _():
        fast(_NCHK)


def _knn(pos, batch):
    posT3 = pos.T.reshape(3, _NCHK, _CW).transpose(1, 0, 2)
    batch2d = batch.reshape(N, 1)
    batchT = batch.reshape(1, N)
    batchT3 = batch.reshape(_NCHK, 1, _CW)
    return pl.pallas_call(
        _knn_body,
        grid=(N // _BR,),
        in_specs=[
            pl.BlockSpec((_BR, 3), lambda i: (i, 0)),
            pl.BlockSpec((_NCHK, 3, _CW), lambda i: (0, 0, 0)),
            pl.BlockSpec((_BR, 1), lambda i: (i, 0)),
            pl.BlockSpec((1, N), lambda i: (0, 0)),
            pl.BlockSpec((_NCHK, 1, _CW), lambda i: (0, 0, 0)),
        ],
        out_specs=pl.BlockSpec((_BR, K), lambda i: (i, 0)),
        out_shape=jax.ShapeDtypeStruct((N, K), jnp.int32),
    )(pos, posT3, batch2d, batchT, batchT3)


_NW = 32
_B = K * N
_BPW = _B // _NW
_CH = 128
_NCH = _BPW // _CH


_D = 128


@jax.jit
def _gather_rows(table, idx3d):
    mesh = plsc.VectorSubcoreMesh(core_axis_name="c", subcore_axis_name="s")

    @functools.partial(
        pl.kernel,
        mesh=mesh,
        out_type=jax.ShapeDtypeStruct((_B, _D), jnp.float32),
        scratch_types=[
            pltpu.VMEM((_NCH, _CH), jnp.int32),
            pltpu.VMEM((4, _CH, _D), jnp.float32),
            [pltpu.SemaphoreType.DMA] * 4,
            [pltpu.SemaphoreType.DMA] * 4,
        ],
    )
    def gk(table_hbm, idx_hbm, out_hbm, idx_v, rows_v, gsems, ssems):
        wid = lax.axis_index("s") * 2 + lax.axis_index("c")
        base = wid * _BPW
        pltpu.sync_copy(idx_hbm.at[wid], idx_v)
        nbuf = 4
        gcp = [None] * nbuf
        scp = [None] * nbuf
        for j in range(nbuf):
            gcp[j] = pltpu.async_copy(
                table_hbm.at[idx_v.at[j]], rows_v.at[j], gsems[j])
        for j in range(_NCH):
            bb = j % nbuf
            gcp[bb].wait()
            scp[bb] = pltpu.async_copy(
                rows_v.at[bb], out_hbm.at[pl.ds(base + j * _CH, _CH)],
                ssems[bb])
            nj = j + nbuf
            if nj < _NCH:
                scp[bb].wait()
                gcp[bb] = pltpu.async_copy(
                    table_hbm.at[idx_v.at[nj]], rows_v.at[bb], gsems[bb])
        for j in range(_NCH - nbuf, _NCH):
            scp[j % nbuf].wait()

    return gk(table, idx3d)


_BT = 1024


def _edge_body(*refs, nlayer, d_real, pad_out):
    if nlayer == 3:
        (f_ref, g_ref, w1_ref, b1_ref, w2_ref, b2_ref, w3_ref,
         b3_ref, out_ref) = refs
    else:
        (f_ref, g_ref, w1_ref, b1_ref, w2_ref, b2_ref,
         out_ref) = refs
    f = f_ref[:, :d_real]
    w1 = w1_ref[...]
    acc = None
    for k in range(K):
        xj = g_ref[k][:, :d_real]
        e = jnp.concatenate([f, xj - f], axis=1)
        h = jnp.dot(e, w1, preferred_element_type=jnp.float32) + b1_ref[...]
        h = jnp.maximum(h, 0.0)
        h = jnp.dot(h, w2_ref[...], preferred_element_type=jnp.float32) + b2_ref[...]
        h = jnp.maximum(h, 0.0)
        if nlayer == 3:
            h = jnp.dot(h, w3_ref[...], preferred_element_type=jnp.float32) + b3_ref[...]
            h = jnp.maximum(h, 0.0)
        acc = h if acc is None else jnp.maximum(acc, h)
    if pad_out:
        acc = jnp.concatenate(
            [acc, jnp.zeros((acc.shape[0], _D - acc.shape[1]), jnp.float32)],
            axis=1)
    out_ref[...] = acc


def _edge_conv(f, g, w1, layers, d_real, pad_out):
    b1 = layers[0][1].reshape(1, -1)
    w2, b2 = layers[1]
    b2 = b2.reshape(1, -1)
    extra = []
    if len(layers) == 3:
        w3, b3 = layers[2]
        b3 = b3.reshape(1, -1)
        extra = [w3, b3]
    h_out = _D if pad_out else layers[-1][0].shape[1]
    full = lambda t: (0, 0)
    in_specs = [
        pl.BlockSpec((_BT, _D), lambda t: (t, 0)),
        pl.BlockSpec((K, _BT, _D), lambda t: (0, t, 0)),
        pl.BlockSpec(w1.shape, full),
        pl.BlockSpec(b1.shape, full),
        pl.BlockSpec(w2.shape, full),
        pl.BlockSpec(b2.shape, full),
    ]
    if extra:
        in_specs += [pl.BlockSpec(extra[0].shape, full),
                     pl.BlockSpec(extra[1].shape, full)]
    return pl.pallas_call(
        functools.partial(_edge_body, nlayer=len(layers), d_real=d_real,
                          pad_out=pad_out),
        grid=(N // _BT,),
        in_specs=in_specs,
        out_specs=pl.BlockSpec((_BT, h_out), lambda t: (t, 0)),
        out_shape=jax.ShapeDtypeStruct((N, h_out), jnp.float32),
    )(f, g, w1, b1, w2, b2, *extra)


def _final_body(comb_ref, batchT_ref, wl1_ref, bl1_ref, wl2_ref, bl2_ref,
                wm_refs, out_ref):
    h = jnp.dot(comb_ref[...], wl1_ref[...], preferred_element_type=jnp.float32)
    h = jnp.maximum(h + bl1_ref[...], 0.0)
    h = jnp.dot(h, wl2_ref[...], preferred_element_type=jnp.float32) + bl2_ref[...]
    cls = lax.broadcasted_iota(jnp.int32, (NC, N), 0)
    oneh = (batchT_ref[...] == cls).astype(jnp.float32)
    pool = jnp.dot(oneh, h, preferred_element_type=jnp.float32,
                   precision=lax.Precision.HIGHEST)
    cnt = jnp.sum(oneh, axis=1, keepdims=True)
    m = pool / jnp.maximum(cnt, 1.0)
    nm = len(wm_refs) // 2
    for i in range(nm):
        m = jnp.dot(m, wm_refs[2 * i][...], preferred_element_type=jnp.float32)
        m = m + wm_refs[2 * i + 1][...]
        if i < nm - 1:
            m = jnp.maximum(m, 0.0)
    out_ref[...] = m


def _final(comb, batch, lin1, mlp):
    batchT = batch.reshape(1, N)
    wl1, bl1 = lin1[0]
    wl2, bl2 = lin1[1]
    args = [comb, batchT, wl1, bl1.reshape(1, -1), wl2, bl2.reshape(1, -1)]
    for w, b in mlp:
        args += [w, b.reshape(1, -1)]

    def body(*refs):
        _final_body(refs[0], refs[1], refs[2], refs[3], refs[4], refs[5],
                    refs[6:-1], refs[-1])

    return pl.pallas_call(
        body,
        out_shape=jax.ShapeDtypeStruct((NC, CLA), jnp.float32),
    )(*args)


def kernel(x, pos, tq, params, batch):
    del tq
    idx = _knn(pos, batch)
    idx3d = idx.T.reshape(_NW, _NCH, _CH)

    xx = jnp.concatenate(
        [x, pos, jnp.zeros((N, _D - 11), jnp.float32)], axis=1)
    g1 = _gather_rows(xx, idx3d).reshape(K, N, _D)
    x1 = _edge_conv(xx, g1, params['conv1'][0][0], params['conv1'], 11, True)

    g2 = _gather_rows(x1, idx3d).reshape(K, N, _D)
    x2 = _edge_conv(x1, g2, params['conv2'][0][0], params['conv2'], 64, True)

    g3 = _gather_rows(x2, idx3d).reshape(K, N, _D)
    x3 = _edge_conv(x2, g3, params['conv2'][0][0], params['conv2'], 64, True)

    g4 = _gather_rows(x3, idx3d).reshape(K, N, _D)
    x4 = _edge_conv(x3, g4, params['conv3'][0][0], params['conv3'], 64, False)

    comb = jnp.concatenate(
        [x1[:, :64], x2[:, :64], x3[:, :64], x4], axis=1)
    return _final(comb, batch, params['lin1'], params['mlp'])
